# Initial kernel scaffold; baseline (speedup 1.0000x reference)
import jax
import jax.numpy as jnp
from jax import lax
from jax.experimental import pallas as pl
from jax.experimental.pallas import tpu as pltpu
from jax.experimental.pallas import tpu_sc as plsc

N = 100000
D = 128
G = 64
NC = 2
NS = 16
NW = NC * NS
ROWS_PER_W = N // NW
CHUNK = 125
NCHUNK = ROWS_PER_W // CHUNK


def _sc_segment_sum(feat, batch3, z128, z16, ones_h):
    mesh = plsc.VectorSubcoreMesh(
        core_axis_name="c", subcore_axis_name="s", num_cores=NC,
        num_subcores=NS)

    @pl.kernel(
        out_type=[
            jax.ShapeDtypeStruct((NC, G, D), jnp.float32),
            jax.ShapeDtypeStruct((NC, G, 16), jnp.float32),
        ],
        mesh=mesh,
        scratch_types=[
            pltpu.VMEM((NCHUNK, CHUNK), jnp.int32),
            pltpu.VMEM((CHUNK, D), jnp.float32),
            pltpu.VMEM((CHUNK, 16), jnp.float32),
            pltpu.VMEM((G, D), jnp.float32),
            pltpu.VMEM((G, 16), jnp.float32),
            pltpu.VMEM_SHARED((G, D), jnp.float32),
            pltpu.VMEM_SHARED((G, 16), jnp.float32),
        ],
    )
    def k(feat_h, batch_h, z128_h, z16_h, ones_hbm, sums_out, cnts_out,
          idx_v, buf, ones_v, tmp, tmp16, acc, cnt):
        c = lax.axis_index("c")
        s = lax.axis_index("s")
        wid = c * NS + s
        pltpu.sync_copy(batch_h.at[wid], idx_v)
        pltpu.sync_copy(ones_hbm, ones_v)

        @pl.when(s == 0)
        def _():
            pltpu.sync_copy(z128_h, tmp)
            pltpu.sync_copy(tmp, acc)
            pltpu.sync_copy(z16_h, tmp16)
            pltpu.sync_copy(tmp16, cnt)

        plsc.subcore_barrier()

        def step(i, carry):
            base = wid * ROWS_PER_W + i * CHUNK
            pltpu.sync_copy(feat_h.at[pl.ds(base, CHUNK)], buf)
            pltpu.sync_copy(buf, acc.at[idx_v.at[i]], add=True)
            pltpu.sync_copy(ones_v, cnt.at[idx_v.at[i]], add=True)
            return carry

        lax.fori_loop(0, NCHUNK, step, 0)
        plsc.subcore_barrier()

        @pl.when(s == 0)
        def _():
            pltpu.sync_copy(acc, tmp)
            pltpu.sync_copy(tmp, sums_out.at[c])
            pltpu.sync_copy(cnt, tmp16)
            pltpu.sync_copy(tmp16, cnts_out.at[c])

    return k(feat, batch3, z128, z16, ones_h)


def _mlp_body(sums_ref, cnts_ref, w1_ref, b1_ref, w2_ref, b2_ref, out_ref):
    sums = sums_ref[0] + sums_ref[1]
    cnt = cnts_ref[0, :, 0] + cnts_ref[1, :, 0]
    mean = sums / jnp.maximum(cnt, 1.0)[:, None]
    h = jnp.dot(mean, w1_ref[...], precision=lax.Precision.HIGHEST,
                preferred_element_type=jnp.float32) + b1_ref[0]
    h = jax.nn.selu(h)
    out_ref[...] = jnp.dot(h, w2_ref[...], precision=lax.Precision.HIGHEST,
                           preferred_element_type=jnp.float32) + b2_ref[0]


def kernel(node_invariant_features, batch, W1, b1, W2, b2):
    feat = node_invariant_features.astype(jnp.float32)
    batch3 = batch.astype(jnp.int32).reshape(NW, NCHUNK, CHUNK)
    z128 = jnp.zeros((G, D), jnp.float32)
    z16 = jnp.zeros((G, 16), jnp.float32)
    ones_h = jnp.ones((CHUNK, 16), jnp.float32)

    sums, cnts = _sc_segment_sum(feat, batch3, z128, z16, ones_h)

    H = W1.shape[1]
    O = W2.shape[1]
    out = pl.pallas_call(
        _mlp_body,
        out_shape=jax.ShapeDtypeStruct((G, O), jnp.float32),
    )(sums, cnts, W1, b1.reshape(1, H), W2, b2.reshape(1, O))
    return out


# speedup vs baseline: 5.4841x; 5.4841x over previous
import jax
import jax.numpy as jnp
from jax import lax
from jax.experimental import pallas as pl
from jax.experimental.pallas import tpu as pltpu
from jax.experimental.pallas import tpu_sc as plsc

N = 100000
D = 128
G = 64
NC = 2
NS = 16
NW = NC * NS
ROWS_PER_W = N // NW
CHUNK = 125
NCHUNK = ROWS_PER_W // CHUNK


def _sc_segment_sum(feat, batch3, z128, z16, ones_h):
    mesh = plsc.VectorSubcoreMesh(
        core_axis_name="c", subcore_axis_name="s", num_cores=NC,
        num_subcores=NS)

    @pl.kernel(
        out_type=[
            jax.ShapeDtypeStruct((NC, G, D), jnp.float32),
            jax.ShapeDtypeStruct((NC, G, 16), jnp.float32),
        ],
        mesh=mesh,
        compiler_params=pltpu.CompilerParams(use_tc_tiling_on_sc=False),
        scratch_types=[
            pltpu.VMEM((NCHUNK, CHUNK), jnp.int32),
            pltpu.VMEM((CHUNK, D), jnp.float32),
            pltpu.VMEM((CHUNK, 16), jnp.float32),
            pltpu.VMEM((G, D), jnp.float32),
            pltpu.VMEM((G, 16), jnp.float32),
            pltpu.VMEM_SHARED((G, D), jnp.float32),
            pltpu.VMEM_SHARED((G, 16), jnp.float32),
        ],
    )
    def k(feat_h, batch_h, z128_h, z16_h, ones_hbm, sums_out, cnts_out,
          idx_v, buf, ones_v, tmp, tmp16, acc, cnt):
        c = lax.axis_index("c")
        s = lax.axis_index("s")
        wid = c * NS + s
        pltpu.sync_copy(batch_h.at[wid], idx_v)
        pltpu.sync_copy(ones_hbm, ones_v)

        @pl.when(s == 0)
        def _():
            pltpu.sync_copy(z128_h, tmp)
            pltpu.sync_copy(tmp, acc)
            pltpu.sync_copy(z16_h, tmp16)
            pltpu.sync_copy(tmp16, cnt)

        plsc.subcore_barrier()

        def step(i, carry):
            base = wid * ROWS_PER_W + i * CHUNK
            pltpu.sync_copy(feat_h.at[pl.ds(base, CHUNK)], buf)
            pltpu.sync_copy(buf, acc.at[idx_v.at[i]], add=True)
            pltpu.sync_copy(ones_v, cnt.at[idx_v.at[i]], add=True)
            return carry

        lax.fori_loop(0, NCHUNK, step, 0)
        plsc.subcore_barrier()

        @pl.when(s == 0)
        def _():
            pltpu.sync_copy(acc, tmp)
            pltpu.sync_copy(tmp, sums_out.at[c])
            pltpu.sync_copy(cnt, tmp16)
            pltpu.sync_copy(tmp16, cnts_out.at[c])

    return k(feat, batch3, z128, z16, ones_h)


def _mlp_body(sums_ref, cnts_ref, w1_ref, b1_ref, w2_ref, b2_ref, out_ref):
    sums = sums_ref[0] + sums_ref[1]
    cnt = cnts_ref[0, :, 0] + cnts_ref[1, :, 0]
    mean = sums / jnp.maximum(cnt, 1.0)[:, None]
    h = jnp.dot(mean, w1_ref[...], precision=lax.Precision.HIGHEST,
                preferred_element_type=jnp.float32) + b1_ref[0]
    alpha = 1.6732632423543772848170429916717
    scale = 1.0507009873554804934193349852946
    h = scale * jnp.where(h > 0, h, alpha * (jnp.exp(h) - 1.0))
    out_ref[...] = jnp.dot(h, w2_ref[...], precision=lax.Precision.HIGHEST,
                           preferred_element_type=jnp.float32) + b2_ref[0]


def kernel(node_invariant_features, batch, W1, b1, W2, b2):
    feat = node_invariant_features.astype(jnp.float32)
    batch3 = batch.astype(jnp.int32).reshape(NW, NCHUNK, CHUNK)
    z128 = jnp.zeros((G, D), jnp.float32)
    z16 = jnp.zeros((G, 16), jnp.float32)
    ones_h = jnp.ones((CHUNK, 16), jnp.float32)

    sums, cnts = _sc_segment_sum(feat, batch3, z128, z16, ones_h)

    H = W1.shape[1]
    O = W2.shape[1]
    out = pl.pallas_call(
        _mlp_body,
        out_shape=jax.ShapeDtypeStruct((G, O), jnp.float32),
    )(sums, cnts, W1, b1.reshape(1, H), W2, b2.reshape(1, O))
    return out


# speedup vs baseline: 6.7001x; 1.2217x over previous
import jax
import jax.numpy as jnp
from jax import lax
from jax.experimental import pallas as pl
from jax.experimental.pallas import tpu as pltpu
from jax.experimental.pallas import tpu_sc as plsc

N = 100000
D = 128
G = 64
NC = 2
NS = 16
NW = NC * NS
ROWS_PER_W = N // NW
CHUNK = 125
NCHUNK = ROWS_PER_W // CHUNK


def _sc_segment_sum(feat, batch3, z128, z16, ones_h):
    mesh = plsc.VectorSubcoreMesh(
        core_axis_name="c", subcore_axis_name="s", num_cores=NC,
        num_subcores=NS)

    @pl.kernel(
        out_type=[
            jax.ShapeDtypeStruct((NC, G, D), jnp.float32),
            jax.ShapeDtypeStruct((NC, G, 16), jnp.float32),
        ],
        mesh=mesh,
        compiler_params=pltpu.CompilerParams(use_tc_tiling_on_sc=False),
        scratch_types=[
            pltpu.VMEM((NCHUNK, CHUNK), jnp.int32),
            pltpu.VMEM((CHUNK, D), jnp.float32),
            pltpu.VMEM((CHUNK, D), jnp.float32),
            pltpu.VMEM((CHUNK, 16), jnp.float32),
            pltpu.VMEM((G, D), jnp.float32),
            pltpu.VMEM((G, 16), jnp.float32),
            pltpu.VMEM_SHARED((G, D), jnp.float32),
            pltpu.VMEM_SHARED((G, 16), jnp.float32),
            pltpu.SemaphoreType.DMA,
            pltpu.SemaphoreType.DMA,
            pltpu.SemaphoreType.DMA,
        ],
    )
    def k(feat_h, batch_h, z128_h, z16_h, ones_hbm, sums_out, cnts_out,
          idx_v, buf0, buf1, ones_v, tmp, tmp16, acc, cnt,
          ld_sem0, ld_sem1, cnt_sem):
        c = lax.axis_index("c")
        s = lax.axis_index("s")
        wid = c * NS + s
        row0 = wid * ROWS_PER_W
        bufs = [buf0, buf1]
        sems = [ld_sem0, ld_sem1]

        loads = [None] * NCHUNK
        loads[0] = pltpu.async_copy(
            feat_h.at[pl.ds(row0, CHUNK)], buf0, ld_sem0)
        pltpu.sync_copy(batch_h.at[wid], idx_v)
        pltpu.sync_copy(ones_hbm, ones_v)

        @pl.when(s == 0)
        def _():
            pltpu.sync_copy(z128_h, tmp)
            pltpu.sync_copy(tmp, acc)
            pltpu.sync_copy(z16_h, tmp16)
            pltpu.sync_copy(tmp16, cnt)

        plsc.subcore_barrier()

        cnt_scatters = []
        for i in range(NCHUNK):
            if i + 1 < NCHUNK:
                loads[i + 1] = pltpu.async_copy(
                    feat_h.at[pl.ds(row0 + (i + 1) * CHUNK, CHUNK)],
                    bufs[(i + 1) % 2], sems[(i + 1) % 2])
            loads[i].wait()
            cnt_scatters.append(pltpu.async_copy(
                ones_v, cnt.at[idx_v.at[i]], cnt_sem, add=True))
            pltpu.sync_copy(bufs[i % 2], acc.at[idx_v.at[i]], add=True)
        for d in cnt_scatters:
            d.wait()
        plsc.subcore_barrier()

        @pl.when(s == 0)
        def _():
            pltpu.sync_copy(acc, tmp)
            pltpu.sync_copy(tmp, sums_out.at[c])
            pltpu.sync_copy(cnt, tmp16)
            pltpu.sync_copy(tmp16, cnts_out.at[c])

    return k(feat, batch3, z128, z16, ones_h)


def _mlp_body(sums_ref, cnts_ref, w1_ref, b1_ref, w2_ref, b2_ref, out_ref):
    sums = sums_ref[0] + sums_ref[1]
    cnt = cnts_ref[0, :, 0] + cnts_ref[1, :, 0]
    mean = sums / jnp.maximum(cnt, 1.0)[:, None]
    h = jnp.dot(mean, w1_ref[...], precision=lax.Precision.HIGHEST,
                preferred_element_type=jnp.float32) + b1_ref[0]
    alpha = 1.6732632423543772848170429916717
    scale = 1.0507009873554804934193349852946
    h = scale * jnp.where(h > 0, h, alpha * (jnp.exp(h) - 1.0))
    out_ref[...] = jnp.dot(h, w2_ref[...], precision=lax.Precision.HIGHEST,
                           preferred_element_type=jnp.float32) + b2_ref[0]


def kernel(node_invariant_features, batch, W1, b1, W2, b2):
    feat = node_invariant_features.astype(jnp.float32)
    batch3 = batch.astype(jnp.int32).reshape(NW, NCHUNK, CHUNK)
    z128 = jnp.zeros((G, D), jnp.float32)
    z16 = jnp.zeros((G, 16), jnp.float32)
    ones_h = jnp.ones((CHUNK, 16), jnp.float32)

    sums, cnts = _sc_segment_sum(feat, batch3, z128, z16, ones_h)

    H = W1.shape[1]
    O = W2.shape[1]
    out = pl.pallas_call(
        _mlp_body,
        out_shape=jax.ShapeDtypeStruct((G, O), jnp.float32),
    )(sums, cnts, W1, b1.reshape(1, H), W2, b2.reshape(1, O))
    return out
